# baseline (speedup 1.0000x reference)
import jax
import jax.numpy as jnp
from jax.experimental import pallas as pl
from jax.experimental.pallas import tpu as pltpu

_B, _S, _E, _H = 8, 512, 1024, 16
_D = _E // _H
_NL = 2

_TB = (((1,), (1,)), ((), ()))
_TN = (((1,), (0,)), ((), ()))


def _qact(x, s):
    return jnp.clip(jnp.round(x / s), -8.0, 7.0) * s


def _bcast(v):
    return jnp.full((1, 128), v, jnp.float32)


def _dot(a, b, dn):
    return jax.lax.dot_general(a, b, dn, preferred_element_type=jnp.float32)


def _wprep_kernel(w_ref, b_ref, qw_ref, qb_ref):
    w = w_ref[0]
    s = jnp.maximum(jnp.max(jnp.abs(w)) / 7.0, 1e-8)
    qw_ref[0] = jnp.clip(jnp.round(w / s), -7.0, 7.0) * s
    b = b_ref[0]
    sb = jnp.maximum(jnp.max(jnp.abs(b)) / 7.0, 1e-8)
    qb_ref[0] = jnp.clip(jnp.round(b / sb), -7.0, 7.0) * sb


def _absmax_kernel(x_ref, m_ref):
    m_ref[0] = _bcast(jnp.max(jnp.abs(x_ref[0])))


def _proj_kernel(sc_ref, x_ref, wq_ref, wk_ref, wv_ref, qb_ref,
                 q_ref, k_ref, v_ref, pm_ref):
    ix = _qact(x_ref[0], sc_ref[0])
    q = (_dot(ix, wq_ref[0], _TB) + qb_ref[0]) * 0.125
    k = _dot(ix, wk_ref[0], _TB) + qb_ref[1]
    v = _dot(ix, wv_ref[0], _TB) + qb_ref[2]
    q_ref[0] = q
    k_ref[0] = k
    v_ref[0] = v
    pm_ref[0] = jnp.concatenate(
        [_bcast(jnp.max(jnp.abs(q))),
         _bcast(jnp.max(jnp.abs(k))),
         _bcast(jnp.max(jnp.abs(v)))], axis=0)


def _smax_kernel(sc_ref, q_ref, k_ref, m_ref):
    iq = _qact(q_ref[0], sc_ref[0])
    ik = _qact(k_ref[0], sc_ref[1])
    m = jnp.float32(0.0)
    for h in range(_H):
        si = _dot(iq[:, h * _D:(h + 1) * _D], ik[:, h * _D:(h + 1) * _D], _TB)
        m = jnp.maximum(m, jnp.max(jnp.abs(si)))
    m_ref[0] = _bcast(m)


def _pmax_kernel(sc_ref, q_ref, k_ref, e_ref, z_ref, m_ref):
    iq = _qact(q_ref[0], sc_ref[0])
    ik = _qact(k_ref[0], sc_ref[1])
    ss = sc_ref[2]
    mz = jnp.float32(jnp.inf)
    for h in range(_H):
        si = _dot(iq[:, h * _D:(h + 1) * _D], ik[:, h * _D:(h + 1) * _D], _TB)
        lf = jnp.clip(jnp.round(si / ss), -8.0, 7.0)
        e_ref[0, h] = lf.astype(jnp.int8)
        sq = lf * ss
        e = jnp.exp(sq - jnp.max(sq, axis=-1, keepdims=True))
        z = jnp.sum(e, axis=-1, keepdims=True)
        z_ref[0, h] = z
        mz = jnp.minimum(mz, jnp.min(z))
    m_ref[0] = _bcast(1.0 / mz)


def _attn_kernel(sc_ref, e_ref, z_ref, v_ref, o_ref, m_ref):
    iv = _qact(v_ref[0], sc_ref[0])
    ss = sc_ref[2]
    m = jnp.float32(0.0)
    for h in range(_H):
        sq = e_ref[0, h].astype(jnp.float32) * ss
        e = jnp.exp(sq - jnp.max(sq, axis=-1, keepdims=True))
        p = e / z_ref[0, h]
        ip = jnp.clip(jnp.round(p / sc_ref[1]), -8.0, 7.0) * sc_ref[1]
        oh = _dot(ip, iv[:, h * _D:(h + 1) * _D], _TN)
        o_ref[0, :, h * _D:(h + 1) * _D] = oh
        m = jnp.maximum(m, jnp.max(jnp.abs(oh)))
    m_ref[0] = _bcast(m)


def _oproj_kernel(sc_ref, x_ref, wo_ref, qb_ref, o_ref, m_ref):
    io = _qact(x_ref[0], sc_ref[0])
    out = _dot(io, wo_ref[0], _TB) + qb_ref[3]
    o_ref[0] = out
    m_ref[0] = _bcast(jnp.max(jnp.abs(out)))


def _fquant_kernel(sc_ref, x_ref, o_ref):
    o_ref[0] = _qact(x_ref[0], sc_ref[0])


def _smem_spec():
    return pl.BlockSpec(memory_space=pltpu.SMEM)


def _cp(vmem=None):
    kw = {"dimension_semantics": ("parallel", "arbitrary")}
    if vmem is not None:
        kw["vmem_limit_bytes"] = vmem
    return pltpu.CompilerParams(**kw)


def kernel(x, Wq, Wk, Wv, Wo, bq, bk, bv, bo):
    f32 = jnp.float32
    Wstk = jnp.stack([Wq, Wk, Wv, Wo])
    bstk = jnp.stack([bq, bk, bv, bo])[:, None, :]

    qW, qb = pl.pallas_call(
        _wprep_kernel,
        grid=(2, 2),
        in_specs=[pl.BlockSpec((1, _E, _E), lambda c, i: (c * 2 + i, 0, 0)),
                  pl.BlockSpec((1, 1, _E), lambda c, i: (c * 2 + i, 0, 0))],
        out_specs=[pl.BlockSpec((1, _E, _E), lambda c, i: (c * 2 + i, 0, 0)),
                   pl.BlockSpec((1, 1, _E), lambda c, i: (c * 2 + i, 0, 0))],
        out_shape=[jax.ShapeDtypeStruct((4, _E, _E), f32),
                   jax.ShapeDtypeStruct((4, 1, _E), f32)],
        compiler_params=_cp(),
        name="wprep",
    )(Wstk, bstk)

    mx = pl.pallas_call(
        _absmax_kernel,
        grid=(2, _B // 2),
        in_specs=[pl.BlockSpec((1, _S, _E), lambda c, i: (c * (_B // 2) + i, 0, 0))],
        out_specs=pl.BlockSpec((1, 1, 128), lambda c, i: (c * (_B // 2) + i, 0, 0)),
        out_shape=jax.ShapeDtypeStruct((_B, 1, 128), f32),
        compiler_params=_cp(),
        name="xabsmax",
    )(x)
    sx = jnp.maximum(jnp.max(mx) / 7.0, 1e-8)

    nb = _B // 2
    wspec = pl.BlockSpec((1, _E, _E), lambda c, i: (0, 0, 0))
    wspec1 = pl.BlockSpec((1, _E, _E), lambda c, i: (1, 0, 0))
    wspec2 = pl.BlockSpec((1, _E, _E), lambda c, i: (2, 0, 0))
    wspec3 = pl.BlockSpec((1, _E, _E), lambda c, i: (3, 0, 0))
    bse = pl.BlockSpec((1, _S, _E), lambda c, i: (c * nb + i, 0, 0))
    pmspec = pl.BlockSpec((1, 1, 128), lambda c, i: (c * nb + i, 0, 0))
    qbspec = pl.BlockSpec((4, 1, _E), lambda c, i: (0, 0, 0))
    sse = jax.ShapeDtypeStruct((_B, _S, _E), f32)
    spm = jax.ShapeDtypeStruct((_B, 1, 128), f32)
    vmem = 56 * 1024 * 1024

    xin = x
    out2 = None
    sf = sx
    for _layer in range(_NL):
        q, k, v, pm = pl.pallas_call(
            _proj_kernel,
            grid=(2, nb),
            in_specs=[_smem_spec(), bse, wspec, wspec1, wspec2, qbspec],
            out_specs=[bse, bse, bse,
                       pl.BlockSpec((1, 3, 128), lambda c, i: (c * nb + i, 0, 0))],
            out_shape=[sse, sse, sse,
                       jax.ShapeDtypeStruct((_B, 3, 128), f32)],
            compiler_params=_cp(vmem),
            name="qkv_proj",
        )(jnp.stack([sx]), xin, qW, qW, qW, qb)
        sq_ = jnp.maximum(jnp.max(pm[:, 0]) / 7.0, 1e-8)
        sk_ = jnp.maximum(jnp.max(pm[:, 1]) / 7.0, 1e-8)
        sv_ = jnp.maximum(jnp.max(pm[:, 2]) / 7.0, 1e-8)

        msi = pl.pallas_call(
            _smax_kernel,
            grid=(2, nb),
            in_specs=[_smem_spec(), bse, bse],
            out_specs=pmspec,
            out_shape=spm,
            compiler_params=_cp(vmem),
            name="score_max",
        )(jnp.stack([sq_, sk_]), q, k)
        ss = jnp.maximum(jnp.max(msi) / 7.0, 1e-8)

        pspec = pl.BlockSpec((1, _H, _S, _S), lambda c, i: (c * nb + i, 0, 0, 0))
        zspec = pl.BlockSpec((1, _H, _S, 1), lambda c, i: (c * nb + i, 0, 0, 0))
        enum, zden, mp = pl.pallas_call(
            _pmax_kernel,
            grid=(2, nb),
            in_specs=[_smem_spec(), bse, bse],
            out_specs=[pspec, zspec, pmspec],
            out_shape=[jax.ShapeDtypeStruct((_B, _H, _S, _S), jnp.int8),
                       jax.ShapeDtypeStruct((_B, _H, _S, 1), f32), spm],
            compiler_params=_cp(vmem),
            name="prob_max",
        )(jnp.stack([sq_, sk_, ss]), q, k)
        sp = jnp.maximum(jnp.max(mp) / 7.0, 1e-8)

        att, mo = pl.pallas_call(
            _attn_kernel,
            grid=(2, nb),
            in_specs=[_smem_spec(), pspec, zspec, bse],
            out_specs=[bse, pmspec],
            out_shape=[sse, spm],
            compiler_params=_cp(vmem),
            name="attn_out",
        )(jnp.stack([sv_, sp, ss]), enum, zden, v)
        so = jnp.maximum(jnp.max(mo) / 7.0, 1e-8)

        out2, mf = pl.pallas_call(
            _oproj_kernel,
            grid=(2, nb),
            in_specs=[_smem_spec(), bse, wspec3, qbspec],
            out_specs=[bse, pmspec],
            out_shape=[sse, spm],
            compiler_params=_cp(vmem),
            name="out_proj",
        )(jnp.stack([so]), att, qW, qb)
        sf = jnp.maximum(jnp.max(mf) / 7.0, 1e-8)

        xin = out2
        sx = sf

    return pl.pallas_call(
        _fquant_kernel,
        grid=(2, nb),
        in_specs=[_smem_spec(), bse],
        out_specs=bse,
        out_shape=sse,
        compiler_params=_cp(),
        name="final_quant",
    )(jnp.stack([sf]), out2)


# speedup vs baseline: 1.1035x; 1.1035x over previous
import jax
import jax.numpy as jnp
from jax.experimental import pallas as pl
from jax.experimental.pallas import tpu as pltpu

_B, _S, _E, _H = 8, 512, 1024, 16
_D = _E // _H
_NL = 2

_TB = (((1,), (1,)), ((), ()))
_TN = (((1,), (0,)), ((), ()))


def _qact(x, s):
    return jnp.clip(jnp.round(x / s), -8.0, 7.0) * s


def _bcast(v):
    return jnp.full((1, 128), v, jnp.float32)


def _dot(a, b, dn):
    return jax.lax.dot_general(a, b, dn, preferred_element_type=jnp.float32)


def _wprep_kernel(w_ref, b_ref, qw_ref, qb_ref):
    w = w_ref[0]
    s = jnp.maximum(jnp.max(jnp.abs(w)) / 7.0, 1e-8)
    qw_ref[0] = jnp.clip(jnp.round(w / s), -7.0, 7.0) * s
    b = b_ref[0]
    sb = jnp.maximum(jnp.max(jnp.abs(b)) / 7.0, 1e-8)
    qb_ref[0] = jnp.clip(jnp.round(b / sb), -7.0, 7.0) * sb


def _absmax_kernel(x_ref, m_ref):
    m_ref[0] = _bcast(jnp.max(jnp.abs(x_ref[0])))


def _proj_kernel(sc_ref, x_ref, wq_ref, wk_ref, wv_ref, qb_ref,
                 q_ref, k_ref, v_ref, pm_ref):
    ix = _qact(x_ref[0], sc_ref[0])
    q = (_dot(ix, wq_ref[0], _TB) + qb_ref[0]) * 0.125
    k = _dot(ix, wk_ref[0], _TB) + qb_ref[1]
    v = _dot(ix, wv_ref[0], _TB) + qb_ref[2]
    q_ref[0] = q
    k_ref[0] = k
    v_ref[0] = v
    pm_ref[0] = jnp.concatenate(
        [_bcast(jnp.max(jnp.abs(q))),
         _bcast(jnp.max(jnp.abs(k))),
         _bcast(jnp.max(jnp.abs(v)))], axis=0)


def _smax_kernel(sc_ref, q_ref, k_ref, m_ref):
    iq = _qact(q_ref[0], sc_ref[0])
    ik = _qact(k_ref[0], sc_ref[1])
    m = jnp.float32(0.0)
    for h in range(_H):
        si = _dot(iq[:, h * _D:(h + 1) * _D], ik[:, h * _D:(h + 1) * _D], _TB)
        m = jnp.maximum(m, jnp.max(jnp.abs(si)))
    m_ref[0] = _bcast(m)


def _pmax_kernel(sc_ref, q_ref, k_ref, e0_ref, e1_ref, e2_ref, e3_ref,
                 z_ref, m_ref):
    iq = _qact(q_ref[0], sc_ref[0])
    ik = _qact(k_ref[0], sc_ref[1])
    ss = sc_ref[2]
    erefs = (e0_ref, e1_ref, e2_ref, e3_ref)
    mz = jnp.float32(jnp.inf)
    for h in range(_H):
        si = _dot(iq[:, h * _D:(h + 1) * _D], ik[:, h * _D:(h + 1) * _D], _TB)
        sq = jnp.clip(jnp.round(si / ss), -8.0, 7.0) * ss
        e = jnp.exp(sq - jnp.max(sq, axis=-1, keepdims=True))
        erefs[h // 4][0, h % 4] = e
        z = jnp.sum(e, axis=-1, keepdims=True)
        z_ref[0, h] = z
        mz = jnp.minimum(mz, jnp.min(z))
    m_ref[0] = _bcast(1.0 / mz)


def _attn_kernel(sc_ref, e0_ref, e1_ref, e2_ref, e3_ref, z_ref, v_ref,
                 o_ref, m_ref):
    iv = _qact(v_ref[0], sc_ref[0])
    erefs = (e0_ref, e1_ref, e2_ref, e3_ref)
    m = jnp.float32(0.0)
    for h in range(_H):
        p = erefs[h // 4][0, h % 4] / z_ref[0, h]
        ip = jnp.clip(jnp.round(p / sc_ref[1]), -8.0, 7.0) * sc_ref[1]
        oh = _dot(ip, iv[:, h * _D:(h + 1) * _D], _TN)
        o_ref[0, :, h * _D:(h + 1) * _D] = oh
        m = jnp.maximum(m, jnp.max(jnp.abs(oh)))
    m_ref[0] = _bcast(m)


def _oproj_kernel(sc_ref, x_ref, wo_ref, qb_ref, o_ref, m_ref):
    io = _qact(x_ref[0], sc_ref[0])
    out = _dot(io, wo_ref[0], _TB) + qb_ref[3]
    o_ref[0] = out
    m_ref[0] = _bcast(jnp.max(jnp.abs(out)))


def _fquant_kernel(sc_ref, x_ref, o_ref):
    o_ref[0] = _qact(x_ref[0], sc_ref[0])


def _smem_spec():
    return pl.BlockSpec(memory_space=pltpu.SMEM)


def _cp(vmem=None):
    kw = {"dimension_semantics": ("parallel", "arbitrary")}
    if vmem is not None:
        kw["vmem_limit_bytes"] = vmem
    return pltpu.CompilerParams(**kw)


def kernel(x, Wq, Wk, Wv, Wo, bq, bk, bv, bo):
    f32 = jnp.float32
    Wstk = jnp.stack([Wq, Wk, Wv, Wo])
    bstk = jnp.stack([bq, bk, bv, bo])[:, None, :]

    qW, qb = pl.pallas_call(
        _wprep_kernel,
        grid=(2, 2),
        in_specs=[pl.BlockSpec((1, _E, _E), lambda c, i: (c * 2 + i, 0, 0)),
                  pl.BlockSpec((1, 1, _E), lambda c, i: (c * 2 + i, 0, 0))],
        out_specs=[pl.BlockSpec((1, _E, _E), lambda c, i: (c * 2 + i, 0, 0)),
                   pl.BlockSpec((1, 1, _E), lambda c, i: (c * 2 + i, 0, 0))],
        out_shape=[jax.ShapeDtypeStruct((4, _E, _E), f32),
                   jax.ShapeDtypeStruct((4, 1, _E), f32)],
        compiler_params=_cp(),
        name="wprep",
    )(Wstk, bstk)

    mx = pl.pallas_call(
        _absmax_kernel,
        grid=(2, _B // 2),
        in_specs=[pl.BlockSpec((1, _S, _E), lambda c, i: (c * (_B // 2) + i, 0, 0))],
        out_specs=pl.BlockSpec((1, 1, 128), lambda c, i: (c * (_B // 2) + i, 0, 0)),
        out_shape=jax.ShapeDtypeStruct((_B, 1, 128), f32),
        compiler_params=_cp(),
        name="xabsmax",
    )(x)
    sx = jnp.maximum(jnp.max(mx) / 7.0, 1e-8)

    nb = _B // 2
    wspec = pl.BlockSpec((1, _E, _E), lambda c, i: (0, 0, 0))
    wspec1 = pl.BlockSpec((1, _E, _E), lambda c, i: (1, 0, 0))
    wspec2 = pl.BlockSpec((1, _E, _E), lambda c, i: (2, 0, 0))
    wspec3 = pl.BlockSpec((1, _E, _E), lambda c, i: (3, 0, 0))
    bse = pl.BlockSpec((1, _S, _E), lambda c, i: (c * nb + i, 0, 0))
    pmspec = pl.BlockSpec((1, 1, 128), lambda c, i: (c * nb + i, 0, 0))
    qbspec = pl.BlockSpec((4, 1, _E), lambda c, i: (0, 0, 0))
    sse = jax.ShapeDtypeStruct((_B, _S, _E), f32)
    spm = jax.ShapeDtypeStruct((_B, 1, 128), f32)
    vmem = 56 * 1024 * 1024

    xin = x
    out2 = None
    sf = sx
    for _layer in range(_NL):
        q, k, v, pm = pl.pallas_call(
            _proj_kernel,
            grid=(2, nb),
            in_specs=[_smem_spec(), bse, wspec, wspec1, wspec2, qbspec],
            out_specs=[bse, bse, bse,
                       pl.BlockSpec((1, 3, 128), lambda c, i: (c * nb + i, 0, 0))],
            out_shape=[sse, sse, sse,
                       jax.ShapeDtypeStruct((_B, 3, 128), f32)],
            compiler_params=_cp(vmem),
            name="qkv_proj",
        )(jnp.stack([sx]), xin, qW, qW, qW, qb)
        sq_ = jnp.maximum(jnp.max(pm[:, 0]) / 7.0, 1e-8)
        sk_ = jnp.maximum(jnp.max(pm[:, 1]) / 7.0, 1e-8)
        sv_ = jnp.maximum(jnp.max(pm[:, 2]) / 7.0, 1e-8)

        msi = pl.pallas_call(
            _smax_kernel,
            grid=(2, nb),
            in_specs=[_smem_spec(), bse, bse],
            out_specs=pmspec,
            out_shape=spm,
            compiler_params=_cp(vmem),
            name="score_max",
        )(jnp.stack([sq_, sk_]), q, k)
        ss = jnp.maximum(jnp.max(msi) / 7.0, 1e-8)

        pspec = pl.BlockSpec((1, _H // 4, _S, _S),
                             lambda c, i: (c * nb + i, 0, 0, 0))
        zspec = pl.BlockSpec((1, _H, _S, 1), lambda c, i: (c * nb + i, 0, 0, 0))
        se4 = jax.ShapeDtypeStruct((_B, _H // 4, _S, _S), f32)
        e0, e1, e2, e3, zden, mp = pl.pallas_call(
            _pmax_kernel,
            grid=(2, nb),
            in_specs=[_smem_spec(), bse, bse],
            out_specs=[pspec, pspec, pspec, pspec, zspec, pmspec],
            out_shape=[se4, se4, se4, se4,
                       jax.ShapeDtypeStruct((_B, _H, _S, 1), f32), spm],
            compiler_params=_cp(vmem),
            name="prob_max",
        )(jnp.stack([sq_, sk_, ss]), q, k)
        sp = jnp.maximum(jnp.max(mp) / 7.0, 1e-8)

        att, mo = pl.pallas_call(
            _attn_kernel,
            grid=(2, nb),
            in_specs=[_smem_spec(), pspec, pspec, pspec, pspec, zspec, bse],
            out_specs=[bse, pmspec],
            out_shape=[sse, spm],
            compiler_params=_cp(vmem),
            name="attn_out",
        )(jnp.stack([sv_, sp]), e0, e1, e2, e3, zden, v)
        so = jnp.maximum(jnp.max(mo) / 7.0, 1e-8)

        out2, mf = pl.pallas_call(
            _oproj_kernel,
            grid=(2, nb),
            in_specs=[_smem_spec(), bse, wspec3, qbspec],
            out_specs=[bse, pmspec],
            out_shape=[sse, spm],
            compiler_params=_cp(vmem),
            name="out_proj",
        )(jnp.stack([so]), att, qW, qb)
        sf = jnp.maximum(jnp.max(mf) / 7.0, 1e-8)

        xin = out2
        sx = sf

    return pl.pallas_call(
        _fquant_kernel,
        grid=(2, nb),
        in_specs=[_smem_spec(), bse],
        out_specs=bse,
        out_shape=sse,
        compiler_params=_cp(),
        name="final_quant",
    )(jnp.stack([sf]), out2)
